# Initial kernel scaffold; baseline (speedup 1.0000x reference)
import jax
import jax.numpy as jnp
from jax.experimental import pallas as pl


def kernel(feature_maps, conv_w, conv_b, cls_w, cls_b, reg_w, reg_b, img_size):
    raise NotImplementedError("write your pallas kernel here")


# speedup vs baseline: 1.2564x; 1.2564x over previous
import numpy as np
from math import sqrt

import jax
import jax.numpy as jnp
from jax import lax
from jax.experimental import pallas as pl

_B, _H, _W, _C = 8, 32, 32, 512
_STRIDE = 16
_RATIOS = [0.5, 1.0, 2.0]
_SCALES = [128.0, 256.0, 512.0]
_NA = 9
_HW = _H * _W


def _anchor_consts():
    a = np.zeros((_HW, 4 * _NA), dtype=np.float32)
    for ri, r in enumerate(_RATIOS):
        for si, s in enumerate(_SCALES):
            i = ri * len(_RATIOS) + si
            aw = s / sqrt(r)
            ah = s * sqrt(r)
            a[:, 4 * i + 2] = aw
            a[:, 4 * i + 3] = ah
    wpos = np.arange(_HW, dtype=np.float32) % _W
    hpos = np.arange(_HW, dtype=np.float32) // _W
    for i in range(_NA):
        a[:, 4 * i + 0] = _STRIDE / 2.0 + wpos * _STRIDE
        a[:, 4 * i + 1] = _STRIDE / 2.0 + hpos * _STRIDE
    ash = np.roll(a, -2, axis=1)
    return a, ash


def _rpn_kernel(x_ref, wt_ref, cb_ref, clsw_ref, clsb_ref, regw_ref,
                regb_ref, a_ref, ash_ref, im_ref, cls_ref, prop_ref):
    x = x_ref[0]

    zrow1 = jnp.zeros((1, _C), dtype=jnp.float32)
    zrow_w = jnp.zeros((_W, _C), dtype=jnp.float32)
    wpos = lax.broadcasted_iota(jnp.int32, (_HW, 1), 0) & (_W - 1)
    xl = jnp.where(wpos >= 1, jnp.concatenate([zrow1, x[:-1]], axis=0), 0.0)
    xr = jnp.where(wpos <= _W - 2, jnp.concatenate([x[1:], zrow1], axis=0), 0.0)

    def taps(v):
        up = jnp.concatenate([zrow_w, v[:-_W]], axis=0)
        dn = jnp.concatenate([v[_W:], zrow_w], axis=0)
        return up, v, dn

    acc = jnp.zeros((_HW, _C), dtype=jnp.float32)
    for kx, col in ((0, xl), (1, x), (2, xr)):
        up, mid, dn = taps(col)
        for ky, t in ((0, up), (1, mid), (2, dn)):
            acc = acc + jnp.dot(t, wt_ref[3 * ky + kx],
                                preferred_element_type=jnp.float32)

    f = jnp.maximum(acc + cb_ref[...], 0.0)

    cls = jnp.dot(f, clsw_ref[...], preferred_element_type=jnp.float32)
    cls_ref[0] = cls + clsb_ref[...]

    t = jnp.dot(f, regw_ref[...], preferred_element_type=jnp.float32)
    t = t + regb_ref[...]

    a = a_ref[...]
    ash = ash_ref[...]
    cmod = lax.broadcasted_iota(jnp.int32, (1, 4 * _NA), 1) & 3
    mask_xy = cmod < 2
    pxy = a + t * ash
    t_wh = jnp.concatenate([t[:, 2:], t[:, :2]], axis=1)
    pwh = ash * jnp.exp(t_wh)
    im = im_ref[0]
    lo = jnp.clip(pxy - pwh * 0.5, 0.0, im)
    hi = jnp.clip(pxy + pwh * 0.5, 0.0, im)
    nwh = hi - lo
    ctr = lo + nwh * 0.5
    nwh_at_23 = jnp.concatenate([nwh[:, -2:], nwh[:, :-2]], axis=1)
    prop_ref[0] = jnp.where(mask_xy, ctr, nwh_at_23)


def kernel(feature_maps, conv_w, conv_b, cls_w, cls_b, reg_w, reg_b, img_size):
    x = jnp.transpose(feature_maps, (0, 2, 3, 1)).reshape(_B, _HW, _C)
    wt = jnp.transpose(conv_w, (2, 3, 1, 0)).reshape(9, _C, _C)
    clsw = jnp.transpose(cls_w[:, :, 0, 0], (1, 0))
    regw = jnp.transpose(reg_w[:, :, 0, 0], (1, 0))
    cb = conv_b.reshape(1, _C)
    clsb = cls_b.reshape(1, 2 * _NA)
    regb = reg_b.reshape(1, 4 * _NA)

    a_np, ash_np = _anchor_consts()
    a_const = jnp.asarray(a_np)
    ash_const = jnp.asarray(ash_np)

    imw = img_size[:, 0].astype(jnp.float32)[:, None]
    imh = img_size[:, 1].astype(jnp.float32)[:, None]
    col_even = (np.arange(4 * _NA) % 2 == 0)[None, :]
    im = jnp.where(jnp.asarray(col_even), imw, imh).reshape(_B, 1, 4 * _NA)

    cls_out, prop_out = pl.pallas_call(
        _rpn_kernel,
        grid=(_B,),
        in_specs=[
            pl.BlockSpec((1, _HW, _C), lambda b: (b, 0, 0)),
            pl.BlockSpec((9, _C, _C), lambda b: (0, 0, 0)),
            pl.BlockSpec((1, _C), lambda b: (0, 0)),
            pl.BlockSpec((_C, 2 * _NA), lambda b: (0, 0)),
            pl.BlockSpec((1, 2 * _NA), lambda b: (0, 0)),
            pl.BlockSpec((_C, 4 * _NA), lambda b: (0, 0)),
            pl.BlockSpec((1, 4 * _NA), lambda b: (0, 0)),
            pl.BlockSpec((_HW, 4 * _NA), lambda b: (0, 0)),
            pl.BlockSpec((_HW, 4 * _NA), lambda b: (0, 0)),
            pl.BlockSpec((1, 1, 4 * _NA), lambda b: (b, 0, 0)),
        ],
        out_specs=[
            pl.BlockSpec((1, _HW, 2 * _NA), lambda b: (b, 0, 0)),
            pl.BlockSpec((1, _HW, 4 * _NA), lambda b: (b, 0, 0)),
        ],
        out_shape=[
            jax.ShapeDtypeStruct((_B, _HW, 2 * _NA), jnp.float32),
            jax.ShapeDtypeStruct((_B, _HW, 4 * _NA), jnp.float32),
        ],
    )(x, wt, cb, clsw, clsb, regw, regb, a_const, ash_const, im)

    proposals = prop_out.reshape(_B, _HW * _NA, 4)
    obj_cls_scores = cls_out.reshape(_B, _HW * _NA, 2)
    return proposals, obj_cls_scores
